# baseline (speedup 1.0000x reference)
import functools

import jax
import jax.numpy as jnp
from jax import lax
from jax.experimental import pallas as pl
from jax.experimental.pallas import tpu as pltpu
from jax.experimental.pallas import tpu_sc as plsc

N = 10000
NC = 2
NS = 16
NW = NC * NS
CHUNK = 128
ACC_ROWS = 10240
ROWS_PT = ACC_ROWS // NS
ZCH = ROWS_PT // CHUNK

_f32 = jnp.float32


def _mesh():
    return plsc.VectorSubcoreMesh(core_axis_name="c", subcore_axis_name="s")


def _make_spmm(F, CPT):
    @functools.partial(
        pl.kernel,
        out_type=jax.ShapeDtypeStruct((NC, ACC_ROWS, F), _f32),
        mesh=_mesh(),
        compiler_params=pltpu.CompilerParams(use_tc_tiling_on_sc=(F % 128 == 0)),
        scratch_types=[
            pltpu.VMEM((CPT, CHUNK), jnp.int32),
            pltpu.VMEM((2, CHUNK), jnp.int32),
            pltpu.VMEM((2, CHUNK), jnp.int32),
            pltpu.VMEM((CHUNK, F), _f32),
            pltpu.VMEM((CHUNK, F), _f32),
            pltpu.VMEM_SHARED((ACC_ROWS, F), _f32),
            pltpu.SemaphoreType.DMA,
            pltpu.SemaphoreType.DMA,
        ],
    )
    def spmm(support_hbm, packed_hbm, out_hbm,
             packed_v, rc0, rc1, gbuf0, gbuf1, acc, sem0, sem1):
        c = lax.axis_index("c")
        s = lax.axis_index("s")
        wid = c * NS + s
        pltpu.sync_copy(packed_hbm.at[wid], packed_v)

        zv = jnp.zeros((16,), _f32)

        def zrow(i, carry):
            for j in range(F // 16):
                gbuf0[i, pl.ds(j * 16, 16)] = zv
            return carry

        lax.fori_loop(0, CHUNK, zrow, 0)
        for k in range(ZCH):
            pltpu.sync_copy(gbuf0, acc.at[pl.ds(s * ROWS_PT + k * CHUNK, CHUNK)])
        plsc.subcore_barrier()

        def prep(j, rc, gbuf, sem):
            for k in range(CHUNK // 16):
                pv = packed_v[j, pl.ds(k * 16, 16)]
                rc[0, pl.ds(k * 16, 16)] = lax.shift_right_logical(pv, 14)
                rc[1, pl.ds(k * 16, 16)] = lax.bitwise_and(pv, 16383)
            pltpu.async_copy(support_hbm.at[rc.at[1]], gbuf, sem)

        def _drain(sem, buf):
            pltpu.make_async_copy(support_hbm.at[pl.ds(0, CHUNK)], buf, sem).wait()

        prep(0, rc0, gbuf0, sem0)

        def pair(i, carry):
            j = 2 * i
            prep(j + 1, rc1, gbuf1, sem1)
            _drain(sem0, gbuf0)
            pltpu.sync_copy(gbuf0, acc.at[rc0.at[0]], add=True)
            nxt = jnp.minimum(j + 2, CPT - 1)
            prep(nxt, rc0, gbuf0, sem0)
            _drain(sem1, gbuf1)
            pltpu.sync_copy(gbuf1, acc.at[rc1.at[0]], add=True)
            return carry

        lax.fori_loop(0, CPT // 2, pair, 0)
        _drain(sem0, gbuf0)
        plsc.subcore_barrier()
        pltpu.sync_copy(acc.at[pl.ds(s * ROWS_PT, ROWS_PT)],
                        out_hbm.at[c, pl.ds(s * ROWS_PT, ROWS_PT)])

    return spmm


def _make_hist(CPT):
    @functools.partial(
        pl.kernel,
        out_type=jax.ShapeDtypeStruct((NC, ACC_ROWS), _f32),
        mesh=_mesh(),
        scratch_types=[
            pltpu.VMEM((CPT, CHUNK), jnp.int32),
            pltpu.VMEM((CHUNK,), _f32),
            pltpu.VMEM((ROWS_PT,), _f32),
            pltpu.VMEM_SHARED((ACC_ROWS,), _f32),
        ],
    )
    def hist(row_hbm, out_hbm, row_v, ones_v, zb, acc):
        c = lax.axis_index("c")
        s = lax.axis_index("s")
        wid = c * NS + s
        pltpu.sync_copy(row_hbm.at[wid], row_v)
        ov = jnp.ones((16,), _f32)
        zv = jnp.zeros((16,), _f32)
        for j in range(CHUNK // 16):
            ones_v[pl.ds(j * 16, 16)] = ov
        for j in range(ROWS_PT // 16):
            zb[pl.ds(j * 16, 16)] = zv
        pltpu.sync_copy(zb, acc.at[pl.ds(s * ROWS_PT, ROWS_PT)])
        plsc.subcore_barrier()

        def step(j, carry):
            pltpu.sync_copy(ones_v, acc.at[row_v.at[j]], add=True)
            return carry

        lax.fori_loop(0, CPT, step, 0)
        plsc.subcore_barrier()
        pltpu.sync_copy(acc.at[pl.ds(s * ROWS_PT, ROWS_PT)],
                        out_hbm.at[c, pl.ds(s * ROWS_PT, ROWS_PT)])

    return hist


_CONTRACT_LAST = (((1,), (1,)), ((), ()))


def _mm1_body(x_ref, w_ref, o_ref):
    o_ref[...] = lax.dot_general(x_ref[...], w_ref[...], _CONTRACT_LAST,
                                 preferred_element_type=_f32)


def _dense1(x, W1):
    return pl.pallas_call(
        _mm1_body,
        grid=(10,),
        in_specs=[pl.BlockSpec((1000, 128), lambda i: (i, 0)),
                  pl.BlockSpec((128, 128), lambda i: (0, 0))],
        out_specs=pl.BlockSpec((1000, 128), lambda i: (i, 0)),
        out_shape=jax.ShapeDtypeStruct((N, 128), _f32),
    )(x, W1)


def _mid_body(p_ref, s1_ref, d0_ref, d1_ref, w2_ref, o_ref):
    inv = 1.0 / (1.0 + d0_ref[...] + d1_ref[...])
    h = jnp.maximum((p_ref[0] + p_ref[1] + s1_ref[...]) * inv, 0.0)
    o_ref[...] = lax.dot_general(h, w2_ref[...], _CONTRACT_LAST,
                                 preferred_element_type=_f32)


def _dense_mid(p, s1, d0, d1, W2):
    return pl.pallas_call(
        _mid_body,
        grid=(10,),
        in_specs=[pl.BlockSpec((2, 1000, 128), lambda i: (0, i, 0)),
                  pl.BlockSpec((1000, 128), lambda i: (i, 0)),
                  pl.BlockSpec((1000, 1), lambda i: (i, 0)),
                  pl.BlockSpec((1000, 1), lambda i: (i, 0)),
                  pl.BlockSpec((64, 128), lambda i: (0, 0))],
        out_specs=pl.BlockSpec((1000, 64), lambda i: (i, 0)),
        out_shape=jax.ShapeDtypeStruct((N, 64), _f32),
    )(p, s1, d0, d1, W2)


def _fin_body(q_ref, s2_ref, d0_ref, d1_ref, o_ref):
    inv = 1.0 / (1.0 + d0_ref[...] + d1_ref[...])
    o_ref[...] = (q_ref[0] + q_ref[1] + s2_ref[...]) * inv


def _dense_fin(q, s2, d0, d1):
    return pl.pallas_call(
        _fin_body,
        grid=(10,),
        in_specs=[pl.BlockSpec((2, 1000, 64), lambda i: (0, i, 0)),
                  pl.BlockSpec((1000, 64), lambda i: (i, 0)),
                  pl.BlockSpec((1000, 1), lambda i: (i, 0)),
                  pl.BlockSpec((1000, 1), lambda i: (i, 0))],
        out_specs=pl.BlockSpec((1000, 64), lambda i: (i, 0)),
        out_shape=jax.ShapeDtypeStruct((N, 64), _f32),
    )(q, s2, d0, d1)


def kernel(x, edge_index, W1, W2):
    E = edge_index.shape[1]
    per_chunk_round = NW * CHUNK
    CPT = -(-E // per_chunk_round)
    CPT += CPT % 2
    EPAD = CPT * per_chunk_round
    pad = EPAD - E
    row = jnp.concatenate([edge_index[0], jnp.full((pad,), N, jnp.int32)])
    col = jnp.concatenate([edge_index[1], jnp.zeros((pad,), jnp.int32)])
    row_r = row.reshape(NW, CPT, CHUNK)
    packed_r = ((row << 14) | col).reshape(NW, CPT, CHUNK)

    deg = _make_hist(CPT)(row_r)
    d0 = deg[0, :N, None]
    d1 = deg[1, :N, None]

    s1 = _dense1(x, W1)
    p = _make_spmm(128, CPT)(s1, packed_r)
    s2 = _dense_mid(p, s1, d0, d1, W2)
    q = _make_spmm(64, CPT)(s2, packed_r)
    return _dense_fin(q, s2, d0, d1)


# speedup vs baseline: 2.6545x; 2.6545x over previous
import functools

import jax
import jax.numpy as jnp
from jax import lax
from jax.experimental import pallas as pl
from jax.experimental.pallas import tpu as pltpu
from jax.experimental.pallas import tpu_sc as plsc

N = 10000
NC = 2
NS = 16
NW = NC * NS
CHUNK = 128
ACC_ROWS = 10240
ROWS_PT = ACC_ROWS // NS
ZCH = ROWS_PT // CHUNK

_f32 = jnp.float32


def _mesh():
    return plsc.VectorSubcoreMesh(core_axis_name="c", subcore_axis_name="s")


def _make_spmm(F, CPT):
    @functools.partial(
        pl.kernel,
        out_type=jax.ShapeDtypeStruct((NC, ACC_ROWS, F), _f32),
        mesh=_mesh(),
        compiler_params=pltpu.CompilerParams(use_tc_tiling_on_sc=(F % 128 == 0)),
        scratch_types=[
            pltpu.VMEM((CPT, CHUNK), jnp.int32),
            pltpu.VMEM((2, CHUNK), jnp.int32),
            pltpu.VMEM((2, CHUNK), jnp.int32),
            pltpu.VMEM((CHUNK, F), _f32),
            pltpu.VMEM((CHUNK, F), _f32),
            pltpu.VMEM_SHARED((ACC_ROWS, F), _f32),
            pltpu.SemaphoreType.DMA,
            pltpu.SemaphoreType.DMA,
        ],
    )
    def spmm(support_hbm, packed_hbm, out_hbm,
             packed_v, rc0, rc1, gbuf0, gbuf1, acc, sem0, sem1):
        c = lax.axis_index("c")
        s = lax.axis_index("s")
        wid = c * NS + s
        pltpu.sync_copy(packed_hbm.at[wid], packed_v)

        zv = jnp.zeros((16,), _f32)

        def zrow(i, carry):
            for j in range(F // 16):
                gbuf0[i, pl.ds(j * 16, 16)] = zv
            return carry

        lax.fori_loop(0, CHUNK, zrow, 0)
        for k in range(ZCH):
            pltpu.sync_copy(gbuf0, acc.at[pl.ds(s * ROWS_PT + k * CHUNK, CHUNK)])
        plsc.subcore_barrier()

        def prep(j, rc, gbuf, sem):
            for k in range(CHUNK // 16):
                pv = packed_v[j, pl.ds(k * 16, 16)]
                rc[0, pl.ds(k * 16, 16)] = lax.shift_right_logical(pv, 14)
                rc[1, pl.ds(k * 16, 16)] = lax.bitwise_and(pv, 16383)
            pltpu.async_copy(support_hbm.at[rc.at[1]], gbuf, sem)

        def _drain(sem, buf):
            pltpu.make_async_copy(support_hbm.at[pl.ds(0, CHUNK)], buf, sem).wait()

        prep(0, rc0, gbuf0, sem0)

        def pair(i, carry):
            j = 2 * i
            prep(j + 1, rc1, gbuf1, sem1)
            _drain(sem0, gbuf0)
            pltpu.sync_copy(gbuf0, acc.at[rc0.at[0]], add=True)
            nxt = jnp.minimum(j + 2, CPT - 1)
            prep(nxt, rc0, gbuf0, sem0)
            _drain(sem1, gbuf1)
            pltpu.sync_copy(gbuf1, acc.at[rc1.at[0]], add=True)
            return carry

        lax.fori_loop(0, CPT // 2, pair, 0)
        _drain(sem0, gbuf0)
        plsc.subcore_barrier()
        pltpu.sync_copy(acc.at[pl.ds(s * ROWS_PT, ROWS_PT)],
                        out_hbm.at[c, pl.ds(s * ROWS_PT, ROWS_PT)])

    return spmm


def _make_hist(CPT):
    @functools.partial(
        pl.kernel,
        out_type=jax.ShapeDtypeStruct((NC, ACC_ROWS), _f32),
        mesh=_mesh(),
        scratch_types=[
            pltpu.VMEM((CPT, CHUNK), jnp.int32),
            pltpu.VMEM((CHUNK,), _f32),
            pltpu.VMEM((ROWS_PT,), _f32),
            pltpu.VMEM_SHARED((ACC_ROWS,), _f32),
        ],
    )
    def hist(row_hbm, out_hbm, row_v, ones_v, zb, acc):
        c = lax.axis_index("c")
        s = lax.axis_index("s")
        wid = c * NS + s
        pltpu.sync_copy(row_hbm.at[wid], row_v)
        ov = jnp.ones((16,), _f32)
        zv = jnp.zeros((16,), _f32)
        for j in range(CHUNK // 16):
            ones_v[pl.ds(j * 16, 16)] = ov
        for j in range(ROWS_PT // 16):
            zb[pl.ds(j * 16, 16)] = zv
        pltpu.sync_copy(zb, acc.at[pl.ds(s * ROWS_PT, ROWS_PT)])
        plsc.subcore_barrier()

        def step(j, carry):
            pltpu.sync_copy(ones_v, acc.at[row_v.at[j]], add=True)
            return carry

        lax.fori_loop(0, CPT, step, 0)
        plsc.subcore_barrier()
        pltpu.sync_copy(acc.at[pl.ds(s * ROWS_PT, ROWS_PT)],
                        out_hbm.at[c, pl.ds(s * ROWS_PT, ROWS_PT)])

    return hist


_CONTRACT_LAST = (((1,), (1,)), ((), ()))


def _mm1_body(x_ref, w_ref, o_ref):
    o_ref[...] = lax.dot_general(x_ref[...], w_ref[...], _CONTRACT_LAST,
                                 preferred_element_type=_f32)


def _dense1(x, W1):
    return pl.pallas_call(
        _mm1_body,
        grid=(10,),
        in_specs=[pl.BlockSpec((1000, 128), lambda i: (i, 0)),
                  pl.BlockSpec((128, 128), lambda i: (0, 0))],
        out_specs=pl.BlockSpec((1000, 128), lambda i: (i, 0)),
        out_shape=jax.ShapeDtypeStruct((N, 128), _f32),
    )(x, W1)


def _mid_body(p_ref, s1_ref, d0_ref, d1_ref, w2_ref, o_ref):
    inv = 1.0 / (1.0 + d0_ref[...] + d1_ref[...])
    h = jnp.maximum((p_ref[0] + p_ref[1] + s1_ref[...]) * inv, 0.0)
    o_ref[...] = lax.dot_general(h, w2_ref[...], _CONTRACT_LAST,
                                 preferred_element_type=_f32)


def _dense_mid(p, s1, d0, d1, W2):
    return pl.pallas_call(
        _mid_body,
        grid=(10,),
        in_specs=[pl.BlockSpec((2, 1000, 128), lambda i: (0, i, 0)),
                  pl.BlockSpec((1000, 128), lambda i: (i, 0)),
                  pl.BlockSpec((1000, 1), lambda i: (i, 0)),
                  pl.BlockSpec((1000, 1), lambda i: (i, 0)),
                  pl.BlockSpec((64, 128), lambda i: (0, 0))],
        out_specs=pl.BlockSpec((1000, 64), lambda i: (i, 0)),
        out_shape=jax.ShapeDtypeStruct((N, 64), _f32),
    )(p, s1, d0, d1, W2)


def _fin_body(q_ref, s2_ref, d0_ref, d1_ref, o_ref):
    inv = 1.0 / (1.0 + d0_ref[...] + d1_ref[...])
    o_ref[...] = (q_ref[0] + q_ref[1] + s2_ref[...]) * inv


def _dense_fin(q, s2, d0, d1):
    return pl.pallas_call(
        _fin_body,
        grid=(10,),
        in_specs=[pl.BlockSpec((2, 1000, 64), lambda i: (0, i, 0)),
                  pl.BlockSpec((1000, 64), lambda i: (i, 0)),
                  pl.BlockSpec((1000, 1), lambda i: (i, 0)),
                  pl.BlockSpec((1000, 1), lambda i: (i, 0))],
        out_specs=pl.BlockSpec((1000, 64), lambda i: (i, 0)),
        out_shape=jax.ShapeDtypeStruct((N, 64), _f32),
    )(q, s2, d0, d1)


def kernel(x, edge_index, W1, W2):
    E = edge_index.shape[1]
    per_chunk_round = NW * CHUNK
    CPT = -(-E // per_chunk_round)
    CPT += CPT % 2
    EPAD = CPT * per_chunk_round
    pad = EPAD - E
    pidx = jnp.arange(pad, dtype=jnp.int32)
    row = jnp.concatenate([edge_index[0], N + pidx % (ACC_ROWS - N)])
    col = jnp.concatenate([edge_index[1], pidx % N])
    row_r = row.reshape(NW, CPT, CHUNK)
    packed_r = ((row << 14) | col).reshape(NW, CPT, CHUNK)

    deg = _make_hist(CPT)(row_r)
    d0 = deg[0, :N, None]
    d1 = deg[1, :N, None]

    s1 = _dense1(x, W1)
    p = _make_spmm(128, CPT)(s1, packed_r)
    s2 = _dense_mid(p, s1, d0, d1, W2)
    q = _make_spmm(64, CPT)(s2, packed_r)
    return _dense_fin(q, s2, d0, d1)
